# Initial kernel scaffold; baseline (speedup 1.0000x reference)
import jax
import jax.numpy as jnp
from jax.experimental import pallas as pl


def kernel(input_ids, token_type_ids, word_table, tt_table, pos_table, gamma, beta):
    raise NotImplementedError("write your pallas kernel here")


# speedup vs baseline: 3.2750x; 3.2750x over previous
import functools

import jax
import jax.numpy as jnp
from jax import lax
from jax.experimental import pallas as pl
from jax.experimental.pallas import tpu as pltpu
from jax.experimental.pallas import tpu_sc as plsc

VOCAB = 100000
EMB = 128
TT_VOCAB = 16
B = 1024
S = 200
EPS = 1e-5

NC, NS, L = 2, 16, 16
NW = NC * NS
N = B * S
PER_W = N // NW
G = 64
NBUF = 4
NSTEP = PER_W // G
NJ = EMB // L


def _rsqrt(x):
    i = lax.bitcast_convert_type(x, jnp.int32)
    i = 0x5F3759DF - lax.shift_right_arithmetic(i, 1)
    y = lax.bitcast_convert_type(i, jnp.float32)
    half = 0.5 * x
    for _ in range(3):
        y = y * (1.5 - half * y * y)
    return y


def _body(ids_hbm, tt_hbm, word_hbm, tt_tab_hbm, pos_hbm, gamma_hbm, beta_hbm,
          out_hbm, ids_v, ttv, pos_v, tt_tab_v, gam_v, bet_v, rowbuf, obuf,
          gsem, osem):
    cid = lax.axis_index("c")
    sid = lax.axis_index("s")
    wid = sid * NC + cid
    base = wid * PER_W

    pltpu.sync_copy(ids_hbm.at[pl.ds(base, PER_W)], ids_v)
    pltpu.sync_copy(tt_hbm.at[pl.ds(base, PER_W)], ttv.at[pl.ds(0, PER_W)])
    pltpu.sync_copy(pos_hbm.at[pl.ds(0, S)], pos_v)
    pltpu.sync_copy(tt_tab_hbm, tt_tab_v)
    pltpu.sync_copy(gamma_hbm, gam_v)
    pltpu.sync_copy(beta_hbm, bet_v)

    gam = [gam_v[pl.ds(L * j, L)] for j in range(NJ)]
    bet = [bet_v[pl.ds(L * j, L)] for j in range(NJ)]

    def gather_start(g, slot):
        idx = ids_v.at[pl.ds(g * G, G)]
        pltpu.make_async_copy(word_hbm.at[idx], rowbuf.at[slot],
                              gsem.at[slot]).start()

    def gather_wait(slot):
        pltpu.make_async_copy(
            word_hbm.at[ids_v.at[pl.ds(0, G)]], rowbuf.at[slot],
            gsem.at[slot]).wait()

    def out_start(g, slot):
        pltpu.make_async_copy(obuf.at[slot],
                              out_hbm.at[pl.ds(base + g * G, G)],
                              osem.at[slot]).start()

    def out_wait(g, slot):
        pltpu.make_async_copy(obuf.at[slot],
                              out_hbm.at[pl.ds(base + g * G, G)],
                              osem.at[slot]).wait()

    for b in range(NBUF):
        gather_start(b, b)

    def step(o, b):
        g = o * NBUF + b

        def token(t, _):
            row = rowbuf.at[b]
            tt = ttv[pl.ds(g * G + t, L)][0]
            s = lax.rem(g * G + t, S)
            acc = []
            for j in range(NJ):
                v = (row[t, pl.ds(L * j, L)]
                     + tt_tab_v[tt, pl.ds(L * j, L)]
                     + pos_v[s, pl.ds(L * j, L)])
                acc.append(v)
            tot = acc[0] + acc[1]
            for j in range(2, NJ):
                tot = tot + acc[j]
            sq = acc[0] * acc[0] + acc[1] * acc[1]
            for j in range(2, NJ):
                sq = sq + acc[j] * acc[j]
            mean = jnp.sum(tot, axis=0) * (1.0 / EMB)
            var = jnp.sum(sq, axis=0) * (1.0 / EMB) - mean * mean
            inv = _rsqrt(var + EPS)
            for j in range(NJ):
                obuf[b, t, pl.ds(L * j, L)] = (acc[j] - mean) * (inv * gam[j]) \
                    + bet[j]
            return 0

        lax.fori_loop(0, G, token, 0, unroll=2)

    def outer(o, _):
        for b in range(NBUF):
            g = o * NBUF + b
            gather_wait(b)

            @pl.when(o > 0)
            def _():
                out_wait((o - 1) * NBUF + b, b)

            step(o, b)
            out_start(g, b)

            @pl.when(o < NSTEP // NBUF - 1)
            def _():
                gather_start(g + NBUF, b)
        return 0

    lax.fori_loop(0, NSTEP // NBUF, outer, 0)

    for b in range(NBUF):
        out_wait(NSTEP - NBUF + b, b)


@jax.jit
def _run(ids, ttids, word_table, tt_tab, pos_tab, gamma, beta):
    k = pl.kernel(
        _body,
        out_type=jax.ShapeDtypeStruct((N, EMB), jnp.float32),
        mesh=plsc.VectorSubcoreMesh(core_axis_name="c", subcore_axis_name="s"),
        compiler_params=pltpu.CompilerParams(needs_layout_passes=False),
        scratch_types=[
            pltpu.VMEM((PER_W,), jnp.int32),
            pltpu.VMEM((PER_W + L,), jnp.int32),
            pltpu.VMEM((S, EMB), jnp.float32),
            pltpu.VMEM((TT_VOCAB, EMB), jnp.float32),
            pltpu.VMEM((EMB,), jnp.float32),
            pltpu.VMEM((EMB,), jnp.float32),
            pltpu.VMEM((NBUF, G, EMB), jnp.float32),
            pltpu.VMEM((NBUF, G, EMB), jnp.float32),
            pltpu.SemaphoreType.DMA((NBUF,)),
            pltpu.SemaphoreType.DMA((NBUF,)),
        ],
    )
    return k(ids, ttids, word_table, tt_tab, pos_tab, gamma, beta)


def kernel(input_ids, token_type_ids, word_table, tt_table, pos_table, gamma,
           beta):
    ids = input_ids.reshape(-1).astype(jnp.int32)
    tts = token_type_ids.reshape(-1).astype(jnp.int32)
    out = _run(ids, tts, word_table, tt_table, pos_table, gamma, beta)
    return out.reshape(B, S, EMB)


# speedup vs baseline: 3.2972x; 1.0068x over previous
import functools

import jax
import jax.numpy as jnp
from jax import lax
from jax.experimental import pallas as pl
from jax.experimental.pallas import tpu as pltpu
from jax.experimental.pallas import tpu_sc as plsc

VOCAB = 100000
EMB = 128
TT_VOCAB = 16
B = 1024
S = 200
EPS = 1e-5

NC, NS, L = 2, 16, 16
NW = NC * NS
N = B * S
PER_W = N // NW
G = 64
NBUF = 4
NSTEP = PER_W // G
NJ = EMB // L


def _rsqrt(x):
    i = lax.bitcast_convert_type(x, jnp.int32)
    i = 0x5F3759DF - lax.shift_right_arithmetic(i, 1)
    y = lax.bitcast_convert_type(i, jnp.float32)
    half = 0.5 * x
    for _ in range(3):
        y = y * (1.5 - half * y * y)
    return y


def _body(ids_hbm, tt_hbm, word_hbm, tt_tab_hbm, pos_hbm, gamma_hbm, beta_hbm,
          out_hbm, ids_v, ttv, pos_v, tt_tab_v, gam_v, bet_v, rowbuf, obuf,
          gsem, osem):
    cid = lax.axis_index("c")
    sid = lax.axis_index("s")
    wid = sid * NC + cid
    base = wid * PER_W

    pltpu.sync_copy(ids_hbm.at[pl.ds(base, PER_W)], ids_v)
    pltpu.sync_copy(tt_hbm.at[pl.ds(base, PER_W)], ttv.at[pl.ds(0, PER_W)])
    pltpu.sync_copy(pos_hbm.at[pl.ds(0, S)], pos_v)
    pltpu.sync_copy(tt_tab_hbm, tt_tab_v)
    pltpu.sync_copy(gamma_hbm, gam_v)
    pltpu.sync_copy(beta_hbm, bet_v)

    gam = [gam_v[pl.ds(L * j, L)] for j in range(NJ)]
    bet = [bet_v[pl.ds(L * j, L)] for j in range(NJ)]

    def gather_start(g, slot):
        idx = ids_v.at[pl.ds(g * G, G)]
        pltpu.make_async_copy(word_hbm.at[idx], rowbuf.at[slot],
                              gsem.at[slot]).start()

    def gather_wait(slot):
        pltpu.make_async_copy(
            word_hbm.at[ids_v.at[pl.ds(0, G)]], rowbuf.at[slot],
            gsem.at[slot]).wait()

    def out_start(g, slot):
        pltpu.make_async_copy(obuf.at[slot],
                              out_hbm.at[pl.ds(base + g * G, G)],
                              osem.at[slot]).start()

    def out_wait(g, slot):
        pltpu.make_async_copy(obuf.at[slot],
                              out_hbm.at[pl.ds(base + g * G, G)],
                              osem.at[slot]).wait()

    for b in range(NBUF):
        gather_start(b, b)

    def step(o, b):
        g = o * NBUF + b

        def token(t, _):
            row = rowbuf.at[b]
            tt = ttv[pl.ds(g * G + t, L)][0]
            s = lax.rem(g * G + t, S)
            acc = []
            for j in range(NJ):
                v = (row[t, pl.ds(L * j, L)]
                     + tt_tab_v[tt, pl.ds(L * j, L)]
                     + pos_v[s, pl.ds(L * j, L)])
                acc.append(v)
            tot = acc[0] + acc[1]
            for j in range(2, NJ):
                tot = tot + acc[j]
            sq = acc[0] * acc[0] + acc[1] * acc[1]
            for j in range(2, NJ):
                sq = sq + acc[j] * acc[j]
            mean = jnp.sum(tot, axis=0) * (1.0 / EMB)
            var = jnp.sum(sq, axis=0) * (1.0 / EMB) - mean * mean
            inv = _rsqrt(var + EPS)
            for j in range(NJ):
                obuf[b, t, pl.ds(L * j, L)] = (acc[j] - mean) * (inv * gam[j]) \
                    + bet[j]
            return 0

        lax.fori_loop(0, G, token, 0, unroll=4)

    def outer(o, _):
        for b in range(NBUF):
            g = o * NBUF + b
            gather_wait(b)

            @pl.when(o > 0)
            def _():
                out_wait((o - 1) * NBUF + b, b)

            step(o, b)
            out_start(g, b)

            @pl.when(o < NSTEP // NBUF - 1)
            def _():
                gather_start(g + NBUF, b)
        return 0

    lax.fori_loop(0, NSTEP // NBUF, outer, 0)

    for b in range(NBUF):
        out_wait(NSTEP - NBUF + b, b)


@jax.jit
def _run(ids, ttids, word_table, tt_tab, pos_tab, gamma, beta):
    k = pl.kernel(
        _body,
        out_type=jax.ShapeDtypeStruct((N, EMB), jnp.float32),
        mesh=plsc.VectorSubcoreMesh(core_axis_name="c", subcore_axis_name="s"),
        compiler_params=pltpu.CompilerParams(needs_layout_passes=False),
        scratch_types=[
            pltpu.VMEM((PER_W,), jnp.int32),
            pltpu.VMEM((PER_W + L,), jnp.int32),
            pltpu.VMEM((S, EMB), jnp.float32),
            pltpu.VMEM((TT_VOCAB, EMB), jnp.float32),
            pltpu.VMEM((EMB,), jnp.float32),
            pltpu.VMEM((EMB,), jnp.float32),
            pltpu.VMEM((NBUF, G, EMB), jnp.float32),
            pltpu.VMEM((NBUF, G, EMB), jnp.float32),
            pltpu.SemaphoreType.DMA((NBUF,)),
            pltpu.SemaphoreType.DMA((NBUF,)),
        ],
    )
    return k(ids, ttids, word_table, tt_tab, pos_tab, gamma, beta)


def kernel(input_ids, token_type_ids, word_table, tt_table, pos_table, gamma,
           beta):
    ids = input_ids.reshape(-1).astype(jnp.int32)
    tts = token_type_ids.reshape(-1).astype(jnp.int32)
    out = _run(ids, tts, word_table, tt_table, pos_table, gamma, beta)
    return out.reshape(B, S, EMB)
